# baseline (speedup 1.0000x reference)
import functools

import jax
import jax.numpy as jnp
from jax.experimental import pallas as pl
from jax.experimental.pallas import tpu as pltpu

_N = 262144
_L = 128
_R = _N // _L
_C = 8
_RB = _R // _C
_HI_BITS = 0x43000000


def _body(pl_ref, gl_ref, off_ref, gb_ref, plm_ref, glm_ref, out_ref,
          nb_ref, nv_ref, acc_ref):
    i = pl.program_id(0)

    @pl.when(i == 0)
    def _init():
        for j in range(8):
            acc_ref[j] = 0.0

    gl = gl_ref[...]
    p = pl_ref[...]
    pos = gl == 1
    neg = gl == 0
    loss_pos = -jnp.maximum(jnp.log(p), -100.0)
    loss_neg = -jnp.maximum(jnp.log(1.0 - p), -100.0)

    negv = jnp.where(neg, loss_neg, -1.0)
    nv_ref[pl.ds(i * _RB, _RB), :] = negv
    nb_ref[pl.ds(i * _RB, _RB), :] = jax.lax.bitcast_convert_type(negv, jnp.int32)

    posf = pos.astype(jnp.float32)
    negf = neg.astype(jnp.float32)
    boxm = (pos | (gl == 2)).astype(jnp.float32)
    landm = (gl == -1).astype(jnp.float32)

    db = off_ref[...] - gb_ref[...]
    dl = plm_ref[...] - glm_ref[...]

    r4 = jax.lax.broadcasted_iota(jnp.int32, (4 * _L, _L), 0) // 4
    c4 = jax.lax.broadcasted_iota(jnp.int32, (4 * _L, _L), 1)
    s4 = (r4 == c4).astype(jnp.float32)
    r10 = jax.lax.broadcasted_iota(jnp.int32, (10 * _L, _L), 0) // 10
    c10 = jax.lax.broadcasted_iota(jnp.int32, (10 * _L, _L), 1)
    s10 = (r10 == c10).astype(jnp.float32)
    bsum = jax.lax.dot(db * db, s4, precision=jax.lax.Precision.HIGHEST)
    lsum = jax.lax.dot(dl * dl, s10, precision=jax.lax.Precision.HIGHEST)

    acc_ref[0] += jnp.sum(posf)
    acc_ref[1] += jnp.sum(negf)
    acc_ref[2] += jnp.sum(loss_pos * posf)
    acc_ref[3] += jnp.sum(loss_neg * negf)
    acc_ref[4] += jnp.sum(boxm)
    acc_ref[5] += jnp.sum(bsum * boxm)
    acc_ref[6] += jnp.sum(landm)
    acc_ref[7] += jnp.sum(lsum * landm)

    @pl.when(i == _C - 1)
    def _finish():
        n_pos = acc_ref[0]
        n_neg = acc_ref[1]
        k_i = n_pos.astype(jnp.int32)
        bits = nb_ref[...]

        def step(_, carry):
            lo, hi = carry
            mid = (lo + hi) // 2
            cnt = jnp.sum((bits >= mid).astype(jnp.int32))
            ok = cnt >= k_i
            return jnp.where(ok, mid, lo), jnp.where(ok, hi, mid)

        lo, _hi = jax.lax.fori_loop(
            0, 31, step, (jnp.int32(0), jnp.int32(_HI_BITS)))
        t = jax.lax.bitcast_convert_type(lo, jnp.float32)
        gtm = bits > lo
        cnt_gt = jnp.sum(gtm.astype(jnp.float32))
        sum_gt = jnp.sum(jnp.where(gtm, nv_ref[...], 0.0))
        sum_neg_top = sum_gt + (n_pos - cnt_gt) * t

        sum_neg = jnp.where(n_neg > n_pos, sum_neg_top, acc_ref[3])
        k_min = jnp.minimum(n_pos, n_neg)
        cls = (acc_ref[2] + sum_neg) / (n_pos + k_min)
        box = acc_ref[5] / (acc_ref[4] * 4.0) * 0.5
        land = acc_ref[7] / (acc_ref[6] * 10.0) * 0.5
        out_ref[0, 0] = cls + box + land


@functools.partial(jax.jit, static_argnames=("interpret",))
def _run(pl2, gl2, off2, gb2, plm2, glm2, interpret=False):
    return pl.pallas_call(
        _body,
        grid=(_C,),
        in_specs=[
            pl.BlockSpec((_RB, _L), lambda i: (i, 0)),
            pl.BlockSpec((_RB, _L), lambda i: (i, 0)),
            pl.BlockSpec((_RB, 4 * _L), lambda i: (i, 0)),
            pl.BlockSpec((_RB, 4 * _L), lambda i: (i, 0)),
            pl.BlockSpec((_RB, 10 * _L), lambda i: (i, 0)),
            pl.BlockSpec((_RB, 10 * _L), lambda i: (i, 0)),
        ],
        out_specs=pl.BlockSpec(memory_space=pltpu.SMEM),
        out_shape=jax.ShapeDtypeStruct((1, 1), jnp.float32),
        scratch_shapes=[
            pltpu.VMEM((_R, _L), jnp.int32),
            pltpu.VMEM((_R, _L), jnp.float32),
            pltpu.SMEM((8,), jnp.float32),
        ],
        interpret=interpret,
    )(pl2, gl2, off2, gb2, plm2, glm2)


def kernel(pred_label, pred_offset, pred_landmarks, gt_label, gt_boxes,
           gt_landmarks):
    pl2 = pred_label.reshape(_R, _L)
    gl2 = gt_label.reshape(_R, _L)
    off2 = pred_offset.reshape(_R, 4 * _L)
    gb2 = gt_boxes.reshape(_R, 4 * _L)
    plm2 = pred_landmarks.reshape(_R, 10 * _L)
    glm2 = gt_landmarks.reshape(_R, 10 * _L)
    out = _run(pl2, gl2, off2, gb2, plm2, glm2)
    return out[0, 0]
